# baseline (speedup 1.0000x reference)
import dataclasses
import functools

import jax
import jax.numpy as jnp
from jax import lax
from jax.experimental import pallas as pl
from jax.experimental.pallas import tpu as pltpu
from jax.experimental.pallas import tpu_sc as plsc

T = 2048
H = 1024
E = 8
F = 512
LG = 16
NW = 32
TPW = T // NW
BLK = 256
BLKLOG = 8
NB_MAX = (2 * T) // BLK + (E - 1)
NSLOT = NB_MAX * BLK
GIDLEN = 32
CH = 32

_MESH = plsc.VectorSubcoreMesh(core_axis_name="c", subcore_axis_name="s")


def _sc_params():
    cp = pltpu.CompilerParams()
    if "needs_layout_passes" in pltpu.CompilerParams.__dataclass_fields__:
        cp = dataclasses.replace(cp, needs_layout_passes=False)
    return cp


def _wid():
    return lax.axis_index("s") * 2 + lax.axis_index("c")


def _tc_logits(x, wg16):
    def body(x_ref, w_ref, o_ref):
        o_ref[...] = jnp.dot(x_ref[...], w_ref[...],
                             preferred_element_type=jnp.float32)

    return pl.pallas_call(
        body,
        out_shape=jax.ShapeDtypeStruct((T, LG), jnp.float32),
    )(x, wg16)


def _sc_route(logits):
    @functools.partial(
        pl.kernel,
        out_type=[
            jax.ShapeDtypeStruct((T,), jnp.int32),
            jax.ShapeDtypeStruct((T,), jnp.int32),
            jax.ShapeDtypeStruct((NW, 16), jnp.int32),
        ],
        mesh=_MESH,
        compiler_params=_sc_params(),
        scratch_types=[
            pltpu.VMEM((TPW, LG), jnp.float32),
            pltpu.VMEM((TPW,), jnp.int32),
            pltpu.VMEM((TPW,), jnp.int32),
            pltpu.VMEM((16,), jnp.int32),
        ],
    )
    def k(lg_hbm, i0_hbm, i1_hbm, lc_hbm, lg_v, i0_v, i1_v, cnt_v):
        w = _wid()
        base = w * TPW
        pltpu.sync_copy(lg_hbm.at[pl.ds(base, TPW)], lg_v)
        lanes = lax.iota(jnp.int32, 16)
        zi = jnp.zeros((16,), jnp.int32)
        cntvec = zi
        for g in range(TPW // 16):
            def tok(jj, carry):
                i0a, i1a, cnt = carry
                v = lg_v[g * 16 + jj]
                vm = jnp.where(lanes < E, v, -3e38)
                m0 = jnp.max(vm)
                a0 = jnp.min(jnp.where(vm == m0, lanes, 127))
                vm2 = jnp.where(lanes == a0, -3e38, vm)
                m1 = jnp.max(vm2)
                a1 = jnp.min(jnp.where(vm2 == m1, lanes, 127))
                here = lanes == jj
                i0a = jnp.where(here, a0, i0a)
                i1a = jnp.where(here, a1, i1a)
                cnt = (cnt + (lanes == a0).astype(jnp.int32)
                       + (lanes == a1).astype(jnp.int32))
                return i0a, i1a, cnt

            i0a, i1a, cntvec = lax.fori_loop(0, 16, tok, (zi, zi, cntvec))
            sl = pl.ds(g * 16, 16)
            i0_v[sl] = i0a
            i1_v[sl] = i1a
        cnt_v[...] = cntvec

        pltpu.sync_copy(i0_v, i0_hbm.at[pl.ds(base, TPW)])
        pltpu.sync_copy(i1_v, i1_hbm.at[pl.ds(base, TPW)])
        pltpu.sync_copy(cnt_v, lc_hbm.at[w])

    return k(logits)


def _sc_dispatch(x, i0, i1, lc):
    @functools.partial(
        pl.kernel,
        out_type=[
            jax.ShapeDtypeStruct((T,), jnp.int32),
            jax.ShapeDtypeStruct((T,), jnp.int32),
            jax.ShapeDtypeStruct((GIDLEN,), jnp.int32),
            jax.ShapeDtypeStruct((16,), jnp.int32),
            jax.ShapeDtypeStruct((NSLOT, H), jnp.float32),
        ],
        mesh=_MESH,
        compiler_params=_sc_params(),
        scratch_types=[
            pltpu.VMEM((NW, 16), jnp.int32),
            pltpu.VMEM((TPW,), jnp.int32),
            pltpu.VMEM((TPW,), jnp.int32),
            pltpu.VMEM((2, TPW), jnp.int32),
            pltpu.VMEM((16,), jnp.int32),
            pltpu.VMEM((GIDLEN,), jnp.int32),
            pltpu.VMEM((16,), jnp.int32),
            pltpu.VMEM((TPW, H), jnp.float32),
        ],
    )
    def k(i0_hbm, i1_hbm, lc_hbm, x_hbm,
          p0_hbm, p1_hbm, gid_hbm, nact_hbm, xs_hbm,
          lc_v, i0_v, i1_v, p_v, woff_v, gid_v, nact_v, x_v):
        w = _wid()
        base = w * TPW
        pltpu.sync_copy(lc_hbm, lc_v)
        lanes = lax.iota(jnp.int32, 16)
        zero = jnp.zeros((16,), jnp.int32)

        def acc(j, carry):
            cnt, wbase = carry
            row = lc_v[j]
            cnt = cnt + row
            wbase = wbase + row * (j < w).astype(jnp.int32)
            return cnt, wbase

        cnt, wbase = lax.fori_loop(0, NW, acc, (zero, zero))
        nblocks = (cnt + (BLK - 1)) >> BLKLOG
        inc = plsc.cumsum(nblocks)
        excl = inc - nblocks
        nact = jnp.max(inc)
        lastb = nact - 1

        for g in range(GIDLEN // 16):
            bvec = lax.iota(jnp.int32, 16) + (g * 16)
            bb = jnp.minimum(bvec, lastb)
            acc_e = jnp.zeros((16,), jnp.int32)
            for e in range(E):
                ince = jnp.sum(jnp.where(lanes == e, inc, 0))
                acc_e = acc_e + (bb >= ince).astype(jnp.int32)
            gid_v[pl.ds(g * 16, 16)] = acc_e

        nact_v[...] = jnp.full((16,), nact, jnp.int32)

        @pl.when(w == 0)
        def _():
            pltpu.sync_copy(gid_v, gid_hbm)
            pltpu.sync_copy(nact_v, nact_hbm)

        pltpu.sync_copy(i0_hbm.at[pl.ds(base, TPW)], i0_v)
        pltpu.sync_copy(i1_hbm.at[pl.ds(base, TPW)], i1_v)

        woff = (excl << BLKLOG) + wbase
        for g in range(TPW // 16):
            i0row = i0_v[pl.ds(g * 16, 16)]
            i1row = i1_v[pl.ds(g * 16, 16)]

            def tok(jj, carry):
                wof, p0a, p1a = carry
                here = lanes == jj
                e0 = jnp.sum(jnp.where(here, i0row, 0))
                is0 = lanes == e0
                p = jnp.sum(jnp.where(is0, wof, 0))
                wof = wof + is0.astype(jnp.int32)
                p0a = jnp.where(here, p, p0a)
                e1 = jnp.sum(jnp.where(here, i1row, 0))
                is1 = lanes == e1
                q = jnp.sum(jnp.where(is1, wof, 0))
                wof = wof + is1.astype(jnp.int32)
                p1a = jnp.where(here, q, p1a)
                return wof, p0a, p1a

            woff, p0a, p1a = lax.fori_loop(0, 16, tok, (woff, zero, zero))
            p_v[0, pl.ds(g * 16, 16)] = p0a
            p_v[1, pl.ds(g * 16, 16)] = p1a

        pltpu.sync_copy(p_v.at[0], p0_hbm.at[pl.ds(base, TPW)])
        pltpu.sync_copy(p_v.at[1], p1_hbm.at[pl.ds(base, TPW)])
        pltpu.sync_copy(x_hbm.at[pl.ds(base, TPW)], x_v)
        pltpu.sync_copy(x_v, xs_hbm.at[p_v.at[0]])
        pltpu.sync_copy(x_v, xs_hbm.at[p_v.at[1]])

    return k(i0, i1, lc, x)


def _tc_shared(x, s_gate_up, s_down, logits):
    def body(x_ref, gu_ref, dn_ref, lg_ref, o_ref):
        gu = jnp.dot(x_ref[...], gu_ref[...],
                     preferred_element_type=jnp.float32)
        g = gu[:, :F]
        u = gu[:, F:]
        act = (g * jax.nn.sigmoid(g)) * u
        sh = jnp.dot(act, dn_ref[...], preferred_element_type=jnp.float32)
        gate = jax.nn.sigmoid(lg_ref[:, E:E + 1])
        o_ref[...] = sh * gate

    return pl.pallas_call(
        body,
        grid=(T // BLK,),
        in_specs=[
            pl.BlockSpec((BLK, H), lambda i: (i, 0)),
            pl.BlockSpec((H, 2 * F), lambda i: (0, 0)),
            pl.BlockSpec((F, H), lambda i: (0, 0)),
            pl.BlockSpec((BLK, LG), lambda i: (i, 0)),
        ],
        out_specs=pl.BlockSpec((BLK, H), lambda i: (i, 0)),
        out_shape=jax.ShapeDtypeStruct((T, H), jnp.float32),
        compiler_params=pltpu.CompilerParams(
            dimension_semantics=("arbitrary",)),
    )(x, s_gate_up, s_down, logits)


def _tc_gemm(gid, nact, xs, w_gate_up, w_down):
    def body(gid_ref, nact_ref, xs_ref, wgu_ref, wd_ref, ys_ref,
             wgub_ref, wdb_ref, cache_ref):
        b = pl.program_id(0)

        @pl.when(b == 0)
        def _():
            cache_ref[0] = -1

        @pl.when(b < nact_ref[0])
        def _():
            e = gid_ref[b]

            @pl.when(e != cache_ref[0])
            def _():
                wgub_ref[...] = wgu_ref[e].astype(jnp.bfloat16)
                wdb_ref[...] = wd_ref[e].astype(jnp.bfloat16)
                cache_ref[0] = e

            gu = jnp.dot(xs_ref[...].astype(jnp.bfloat16), wgub_ref[...],
                         preferred_element_type=jnp.float32)
            g = gu[:, :F]
            u = gu[:, F:]
            act = ((g * jax.nn.sigmoid(g)) * u).astype(jnp.bfloat16)
            ys_ref[...] = jnp.dot(act, wdb_ref[...],
                                  preferred_element_type=jnp.float32)

    grid_spec = pltpu.PrefetchScalarGridSpec(
        num_scalar_prefetch=2,
        grid=(NB_MAX,),
        in_specs=[
            pl.BlockSpec((BLK, H), lambda b, gid, na: (b, 0)),
            pl.BlockSpec((E, H, 2 * F), lambda b, gid, na: (0, 0, 0)),
            pl.BlockSpec((E, F, H), lambda b, gid, na: (0, 0, 0)),
        ],
        out_specs=pl.BlockSpec((BLK, H), lambda b, gid, na: (b, 0)),
        scratch_shapes=[
            pltpu.VMEM((H, 2 * F), jnp.bfloat16),
            pltpu.VMEM((F, H), jnp.bfloat16),
            pltpu.SMEM((1,), jnp.int32),
        ],
    )
    return pl.pallas_call(
        body,
        grid_spec=grid_spec,
        out_shape=jax.ShapeDtypeStruct((NSLOT, H), jnp.float32),
        compiler_params=pltpu.CompilerParams(
            dimension_semantics=("arbitrary",)),
    )(gid, nact, xs, w_gate_up, w_down)


def _sc_gather(ys, p0, p1):
    @functools.partial(
        pl.kernel,
        out_type=[
            jax.ShapeDtypeStruct((T, H), jnp.float32),
            jax.ShapeDtypeStruct((T, H), jnp.float32),
        ],
        mesh=_MESH,
        compiler_params=_sc_params(),
        scratch_types=[
            pltpu.VMEM((2, TPW), jnp.int32),
            pltpu.VMEM((TPW // 2, H), jnp.float32),
            pltpu.VMEM((TPW // 2, H), jnp.float32),
        ],
    )
    def k(ys_hbm, p0_hbm, p1_hbm, y0_hbm, y1_hbm, pidx_v, r0_v, r1_v):
        w = _wid()
        base = w * TPW
        pltpu.sync_copy(p0_hbm.at[pl.ds(base, TPW)], pidx_v.at[0])
        pltpu.sync_copy(p1_hbm.at[pl.ds(base, TPW)], pidx_v.at[1])
        half = TPW // 2
        for c in range(2):
            pltpu.sync_copy(ys_hbm.at[pidx_v.at[0, pl.ds(c * half, half)]],
                            r0_v)
            pltpu.sync_copy(r0_v, y0_hbm.at[pl.ds(base + c * half, half)])
            pltpu.sync_copy(ys_hbm.at[pidx_v.at[1, pl.ds(c * half, half)]],
                            r1_v)
            pltpu.sync_copy(r1_v, y1_hbm.at[pl.ds(base + c * half, half)])

    return k(ys, p0, p1)


def _tc_fma(y0, y1, logits, shared):
    def body(y0_ref, y1_ref, lg_ref, sh_ref, o_ref):
        lg = lg_ref[...]
        cols = lax.broadcasted_iota(jnp.int32, (BLK, LG), 1)
        vm = jnp.where(cols < E, lg, -3e38)
        m0 = jnp.max(vm, axis=1, keepdims=True)
        first = jnp.min(jnp.where(vm == m0, cols, 127), axis=1,
                        keepdims=True)
        vm2 = jnp.where(cols == first, -3e38, vm)
        m1 = jnp.max(vm2, axis=1, keepdims=True)
        a0 = 1.0 / (1.0 + jnp.exp(m1 - m0))
        a1 = 1.0 - a0
        o_ref[...] = (y0_ref[...] * a0 + y1_ref[...] * a1 + sh_ref[...])

    return pl.pallas_call(
        body,
        grid=(T // BLK,),
        in_specs=[
            pl.BlockSpec((BLK, H), lambda i: (i, 0)),
            pl.BlockSpec((BLK, H), lambda i: (i, 0)),
            pl.BlockSpec((BLK, LG), lambda i: (i, 0)),
            pl.BlockSpec((BLK, H), lambda i: (i, 0)),
        ],
        out_specs=pl.BlockSpec((BLK, H), lambda i: (i, 0)),
        out_shape=jax.ShapeDtypeStruct((T, H), jnp.float32),
        compiler_params=pltpu.CompilerParams(
            dimension_semantics=("arbitrary",)),
    )(y0, y1, logits, shared)


def kernel(hidden_states, W_gate, w_gate_up, w_down, s_gate_up, s_down):
    x = hidden_states
    wg16 = jnp.pad(W_gate, ((0, 0), (0, LG - (E + 1))))
    logits = _tc_logits(x, wg16)
    i0, i1, lc = _sc_route(logits)
    p0, p1, gid, nact, xs = _sc_dispatch(x, i0, i1, lc)
    shared = _tc_shared(x, s_gate_up, s_down, logits)
    ys = _tc_gemm(gid, nact, xs, w_gate_up, w_down)
    y0, y1 = _sc_gather(ys, p0, p1)
    return _tc_fma(y0, y1, logits, shared)


# speedup vs baseline: 1.0263x; 1.0263x over previous
import dataclasses
import functools

import jax
import jax.numpy as jnp
from jax import lax
from jax.experimental import pallas as pl
from jax.experimental.pallas import tpu as pltpu
from jax.experimental.pallas import tpu_sc as plsc

T = 2048
H = 1024
E = 8
F = 512
LG = 16
NW = 32
TPW = T // NW
BLK = 256
BLKLOG = 8
NB_MAX = (2 * T) // BLK + (E - 1)
NSLOT = NB_MAX * BLK
GIDLEN = 32
CH = 32

_MESH = plsc.VectorSubcoreMesh(core_axis_name="c", subcore_axis_name="s")


def _sc_params():
    cp = pltpu.CompilerParams()
    if "needs_layout_passes" in pltpu.CompilerParams.__dataclass_fields__:
        cp = dataclasses.replace(cp, needs_layout_passes=False)
    return cp


def _wid():
    return lax.axis_index("s") * 2 + lax.axis_index("c")


def _tc_logits(x, wg16):
    def body(x_ref, w_ref, o_ref):
        o_ref[...] = jnp.dot(x_ref[...], w_ref[...],
                             preferred_element_type=jnp.float32)

    return pl.pallas_call(
        body,
        out_shape=jax.ShapeDtypeStruct((T, LG), jnp.float32),
    )(x, wg16)


def _sc_route(logits):
    @functools.partial(
        pl.kernel,
        out_type=[
            jax.ShapeDtypeStruct((T,), jnp.int32),
            jax.ShapeDtypeStruct((T,), jnp.int32),
            jax.ShapeDtypeStruct((NW, 16), jnp.int32),
        ],
        mesh=_MESH,
        compiler_params=_sc_params(),
        scratch_types=[
            pltpu.VMEM((TPW, LG), jnp.float32),
            pltpu.VMEM((TPW,), jnp.int32),
            pltpu.VMEM((TPW,), jnp.int32),
            pltpu.VMEM((16,), jnp.int32),
        ],
    )
    def k(lg_hbm, i0_hbm, i1_hbm, lc_hbm, lg_v, i0_v, i1_v, cnt_v):
        w = _wid()
        base = w * TPW
        pltpu.sync_copy(lg_hbm.at[pl.ds(base, TPW)], lg_v)
        lanes = lax.iota(jnp.int32, 16)
        zi = jnp.zeros((16,), jnp.int32)
        cntvec = zi
        for g in range(TPW // 16):
            def tok(jj, carry):
                i0a, i1a, cnt = carry
                v = lg_v[g * 16 + jj]
                vm = jnp.where(lanes < E, v, -3e38)
                m0 = jnp.max(vm)
                a0 = jnp.min(jnp.where(vm == m0, lanes, 127))
                vm2 = jnp.where(lanes == a0, -3e38, vm)
                m1 = jnp.max(vm2)
                a1 = jnp.min(jnp.where(vm2 == m1, lanes, 127))
                here = lanes == jj
                i0a = jnp.where(here, a0, i0a)
                i1a = jnp.where(here, a1, i1a)
                cnt = (cnt + (lanes == a0).astype(jnp.int32)
                       + (lanes == a1).astype(jnp.int32))
                return i0a, i1a, cnt

            i0a, i1a, cntvec = lax.fori_loop(0, 16, tok, (zi, zi, cntvec))
            sl = pl.ds(g * 16, 16)
            i0_v[sl] = i0a
            i1_v[sl] = i1a
        cnt_v[...] = cntvec

        pltpu.sync_copy(i0_v, i0_hbm.at[pl.ds(base, TPW)])
        pltpu.sync_copy(i1_v, i1_hbm.at[pl.ds(base, TPW)])
        pltpu.sync_copy(cnt_v, lc_hbm.at[w])

    return k(logits)


def _sc_dispatch(x, i0, i1, lc):
    @functools.partial(
        pl.kernel,
        out_type=[
            jax.ShapeDtypeStruct((T,), jnp.int32),
            jax.ShapeDtypeStruct((T,), jnp.int32),
            jax.ShapeDtypeStruct((GIDLEN,), jnp.int32),
            jax.ShapeDtypeStruct((16,), jnp.int32),
            jax.ShapeDtypeStruct((NSLOT, H), jnp.float32),
        ],
        mesh=_MESH,
        compiler_params=_sc_params(),
        scratch_types=[
            pltpu.VMEM((NW, 16), jnp.int32),
            pltpu.VMEM((TPW,), jnp.int32),
            pltpu.VMEM((TPW,), jnp.int32),
            pltpu.VMEM((2, TPW), jnp.int32),
            pltpu.VMEM((16,), jnp.int32),
            pltpu.VMEM((GIDLEN,), jnp.int32),
            pltpu.VMEM((16,), jnp.int32),
            pltpu.VMEM((TPW, H), jnp.float32),
        ],
    )
    def k(i0_hbm, i1_hbm, lc_hbm, x_hbm,
          p0_hbm, p1_hbm, gid_hbm, nact_hbm, xs_hbm,
          lc_v, i0_v, i1_v, p_v, woff_v, gid_v, nact_v, x_v):
        w = _wid()
        base = w * TPW
        pltpu.sync_copy(lc_hbm, lc_v)
        lanes = lax.iota(jnp.int32, 16)
        zero = jnp.zeros((16,), jnp.int32)

        def acc(j, carry):
            cnt, wbase = carry
            row = lc_v[j]
            cnt = cnt + row
            wbase = wbase + row * (j < w).astype(jnp.int32)
            return cnt, wbase

        cnt, wbase = lax.fori_loop(0, NW, acc, (zero, zero))
        nblocks = (cnt + (BLK - 1)) >> BLKLOG
        inc = plsc.cumsum(nblocks)
        excl = inc - nblocks
        nact = jnp.max(inc)
        lastb = nact - 1

        for g in range(GIDLEN // 16):
            bvec = lax.iota(jnp.int32, 16) + (g * 16)
            bb = jnp.minimum(bvec, lastb)
            acc_e = jnp.zeros((16,), jnp.int32)
            for e in range(E):
                ince = jnp.sum(jnp.where(lanes == e, inc, 0))
                acc_e = acc_e + (bb >= ince).astype(jnp.int32)
            gid_v[pl.ds(g * 16, 16)] = acc_e

        nact_v[...] = jnp.full((16,), nact, jnp.int32)

        @pl.when(w == 0)
        def _():
            pltpu.sync_copy(gid_v, gid_hbm)
            pltpu.sync_copy(nact_v, nact_hbm)

        pltpu.sync_copy(i0_hbm.at[pl.ds(base, TPW)], i0_v)
        pltpu.sync_copy(i1_hbm.at[pl.ds(base, TPW)], i1_v)

        woff = (excl << BLKLOG) + wbase
        for g in range(TPW // 16):
            i0row = i0_v[pl.ds(g * 16, 16)]
            i1row = i1_v[pl.ds(g * 16, 16)]

            def tok(jj, carry):
                wof, p0a, p1a = carry
                here = lanes == jj
                e0 = jnp.sum(jnp.where(here, i0row, 0))
                is0 = lanes == e0
                p = jnp.sum(jnp.where(is0, wof, 0))
                wof = wof + is0.astype(jnp.int32)
                p0a = jnp.where(here, p, p0a)
                e1 = jnp.sum(jnp.where(here, i1row, 0))
                is1 = lanes == e1
                q = jnp.sum(jnp.where(is1, wof, 0))
                wof = wof + is1.astype(jnp.int32)
                p1a = jnp.where(here, q, p1a)
                return wof, p0a, p1a

            woff, p0a, p1a = lax.fori_loop(0, 16, tok, (woff, zero, zero))
            p_v[0, pl.ds(g * 16, 16)] = p0a
            p_v[1, pl.ds(g * 16, 16)] = p1a

        pltpu.sync_copy(p_v.at[0], p0_hbm.at[pl.ds(base, TPW)])
        pltpu.sync_copy(p_v.at[1], p1_hbm.at[pl.ds(base, TPW)])
        pltpu.sync_copy(x_hbm.at[pl.ds(base, TPW)], x_v)
        pltpu.sync_copy(x_v, xs_hbm.at[p_v.at[0]])
        pltpu.sync_copy(x_v, xs_hbm.at[p_v.at[1]])

    return k(i0, i1, lc, x)


def _tc_shared(x, s_gate_up, s_down, logits):
    def body(x_ref, gu_ref, dn_ref, lg_ref, o_ref):
        gu = jnp.dot(x_ref[...], gu_ref[...],
                     preferred_element_type=jnp.float32)
        g = gu[:, :F]
        u = gu[:, F:]
        act = (g * jax.nn.sigmoid(g)) * u
        sh = jnp.dot(act, dn_ref[...], preferred_element_type=jnp.float32)
        gate = jax.nn.sigmoid(lg_ref[:, E:E + 1])
        o_ref[...] = sh * gate

    return pl.pallas_call(
        body,
        grid=(T // BLK,),
        in_specs=[
            pl.BlockSpec((BLK, H), lambda i: (i, 0)),
            pl.BlockSpec((H, 2 * F), lambda i: (0, 0)),
            pl.BlockSpec((F, H), lambda i: (0, 0)),
            pl.BlockSpec((BLK, LG), lambda i: (i, 0)),
        ],
        out_specs=pl.BlockSpec((BLK, H), lambda i: (i, 0)),
        out_shape=jax.ShapeDtypeStruct((T, H), jnp.float32),
        compiler_params=pltpu.CompilerParams(
            dimension_semantics=("arbitrary",)),
    )(x, s_gate_up, s_down, logits)


def _tc_gemm(gid, nact, xs, w_gate_up, w_down):
    def body(gid_ref, nact_ref, xs_ref, wgu_ref, wd_ref, ys_ref):
        b = pl.program_id(0)

        @pl.when(b < nact_ref[0])
        def _():
            e = gid_ref[b]
            gu = jnp.dot(xs_ref[...], wgu_ref[e],
                         preferred_element_type=jnp.float32)
            g = gu[:, :F]
            u = gu[:, F:]
            act = (g * jax.nn.sigmoid(g)) * u
            ys_ref[...] = jnp.dot(act, wd_ref[e],
                                  preferred_element_type=jnp.float32)

    grid_spec = pltpu.PrefetchScalarGridSpec(
        num_scalar_prefetch=2,
        grid=(NB_MAX,),
        in_specs=[
            pl.BlockSpec((BLK, H), lambda b, gid, na: (b, 0)),
            pl.BlockSpec((E, H, 2 * F), lambda b, gid, na: (0, 0, 0)),
            pl.BlockSpec((E, F, H), lambda b, gid, na: (0, 0, 0)),
        ],
        out_specs=pl.BlockSpec((BLK, H), lambda b, gid, na: (b, 0)),
    )
    return pl.pallas_call(
        body,
        grid_spec=grid_spec,
        out_shape=jax.ShapeDtypeStruct((NSLOT, H), jnp.float32),
        compiler_params=pltpu.CompilerParams(
            dimension_semantics=("parallel",)),
    )(gid, nact, xs, w_gate_up, w_down)


def _sc_gather(ys, p0, p1):
    @functools.partial(
        pl.kernel,
        out_type=[
            jax.ShapeDtypeStruct((T, H), jnp.float32),
            jax.ShapeDtypeStruct((T, H), jnp.float32),
        ],
        mesh=_MESH,
        compiler_params=_sc_params(),
        scratch_types=[
            pltpu.VMEM((2, TPW), jnp.int32),
            pltpu.VMEM((TPW // 2, H), jnp.float32),
            pltpu.VMEM((TPW // 2, H), jnp.float32),
        ],
    )
    def k(ys_hbm, p0_hbm, p1_hbm, y0_hbm, y1_hbm, pidx_v, r0_v, r1_v):
        w = _wid()
        base = w * TPW
        pltpu.sync_copy(p0_hbm.at[pl.ds(base, TPW)], pidx_v.at[0])
        pltpu.sync_copy(p1_hbm.at[pl.ds(base, TPW)], pidx_v.at[1])
        half = TPW // 2
        for c in range(2):
            pltpu.sync_copy(ys_hbm.at[pidx_v.at[0, pl.ds(c * half, half)]],
                            r0_v)
            pltpu.sync_copy(r0_v, y0_hbm.at[pl.ds(base + c * half, half)])
            pltpu.sync_copy(ys_hbm.at[pidx_v.at[1, pl.ds(c * half, half)]],
                            r1_v)
            pltpu.sync_copy(r1_v, y1_hbm.at[pl.ds(base + c * half, half)])

    return k(ys, p0, p1)


def _tc_fma(y0, y1, logits, shared):
    def body(y0_ref, y1_ref, lg_ref, sh_ref, o_ref):
        lg = lg_ref[...]
        cols = lax.broadcasted_iota(jnp.int32, lg.shape, 1)
        vm = jnp.where(cols < E, lg, -3e38)
        m0 = jnp.max(vm, axis=1, keepdims=True)
        first = jnp.min(jnp.where(vm == m0, cols, 127), axis=1,
                        keepdims=True)
        vm2 = jnp.where(cols == first, -3e38, vm)
        m1 = jnp.max(vm2, axis=1, keepdims=True)
        a0 = 1.0 / (1.0 + jnp.exp(m1 - m0))
        a1 = 1.0 - a0
        o_ref[...] = (y0_ref[...] * a0 + y1_ref[...] * a1 + sh_ref[...])

    fb = 2 * BLK
    return pl.pallas_call(
        body,
        grid=(T // fb,),
        in_specs=[
            pl.BlockSpec((fb, H), lambda i: (i, 0)),
            pl.BlockSpec((fb, H), lambda i: (i, 0)),
            pl.BlockSpec((fb, LG), lambda i: (i, 0)),
            pl.BlockSpec((fb, H), lambda i: (i, 0)),
        ],
        out_specs=pl.BlockSpec((fb, H), lambda i: (i, 0)),
        out_shape=jax.ShapeDtypeStruct((T, H), jnp.float32),
        compiler_params=pltpu.CompilerParams(
            dimension_semantics=("parallel",)),
    )(y0, y1, logits, shared)


def kernel(hidden_states, W_gate, w_gate_up, w_down, s_gate_up, s_down):
    x = hidden_states
    wg16 = jnp.pad(W_gate, ((0, 0), (0, LG - (E + 1))))
    logits = _tc_logits(x, wg16)
    i0, i1, lc = _sc_route(logits)
    p0, p1, gid, nact, xs = _sc_dispatch(x, i0, i1, lc)
    shared = _tc_shared(x, s_gate_up, s_down, logits)
    ys = _tc_gemm(gid, nact, xs, w_gate_up, w_down)
    y0, y1 = _sc_gather(ys, p0, p1)
    return _tc_fma(y0, y1, logits, shared)
